# baseline (speedup 1.0000x reference)
import functools

import jax
import jax.numpy as jnp
from jax import lax
from jax.experimental import pallas as pl
from jax.experimental.pallas import tpu as pltpu
from jax.experimental.pallas import tpu_sc as plsc

N = 10000
E = 320000
F = 64
NC = 2
NS = 16
NW = NC * NS
CHUNK = 128
T = 80
E_PAD = NW * CHUNK * T
PER_W = T * CHUNK
N_PAD = ((N + NS - 1) // NS + 7) // 8 * 8 * NS
ROWS_PER_TILE = N_PAD // NS
TRASH = N_PAD - N

NBLK = 1000
EBLK = 4096

@functools.cache
def _mesh():
    return plsc.VectorSubcoreMesh(core_axis_name="c", subcore_axis_name="s")


def _prep_body(x_ref, batch_ref, sel_ref, wsel_ref, bsel_ref, wu_ref,
               wsrc_ref, be0_ref, wcat_ref, bcat_ref,
               tsrc_ref, tcol_ref, u_ref):
    f32 = jnp.float32
    u_val = jnp.dot(sel_ref[...], wsel_ref[...], preferred_element_type=f32) + bsel_ref[...]
    ug = jnp.dot(u_val, wu_ref[...], preferred_element_type=f32)
    xb = x_ref[...]
    gids = lax.broadcasted_iota(jnp.int32, (NBLK, 8), 1)
    onehot = (batch_ref[...] == gids).astype(f32)
    ts = (jnp.dot(xb, wsrc_ref[...], preferred_element_type=f32)
          + jnp.dot(onehot, ug, preferred_element_type=f32)
          + be0_ref[...])
    tsrc_ref[...] = jnp.concatenate([ts, jnp.zeros((NBLK, F), f32)], axis=1)
    tcol_ref[...] = jnp.dot(xb, wcat_ref[...], preferred_element_type=f32) + bcat_ref[...]
    u_ref[...] = u_val


def _prep(x, batch2, selection, W_sel, b_sel2, W_u, W_src, be02, W_cat, bcat2):
    f32 = jnp.float32
    grid = N // NBLK
    return pl.pallas_call(
        _prep_body,
        grid=(grid,),
        in_specs=[
            pl.BlockSpec((NBLK, 9), lambda i: (i, 0)),
            pl.BlockSpec((NBLK, 1), lambda i: (i, 0)),
            pl.BlockSpec((8, 512), lambda i: (0, 0)),
            pl.BlockSpec((512, F), lambda i: (0, 0)),
            pl.BlockSpec((1, F), lambda i: (0, 0)),
            pl.BlockSpec((F, F), lambda i: (0, 0)),
            pl.BlockSpec((9, F), lambda i: (0, 0)),
            pl.BlockSpec((1, F), lambda i: (0, 0)),
            pl.BlockSpec((9, 2 * F), lambda i: (0, 0)),
            pl.BlockSpec((1, 2 * F), lambda i: (0, 0)),
        ],
        out_specs=[
            pl.BlockSpec((NBLK, 2 * F), lambda i: (i, 0)),
            pl.BlockSpec((NBLK, 2 * F), lambda i: (i, 0)),
            pl.BlockSpec((8, F), lambda i: (0, 0)),
        ],
        out_shape=[
            jax.ShapeDtypeStruct((N, 2 * F), f32),
            jax.ShapeDtypeStruct((N, 2 * F), f32),
            jax.ShapeDtypeStruct((8, F), f32),
        ],
    )(x, batch2, selection, W_sel, b_sel2, W_u, W_src, be02, W_cat, bcat2)


_GS = 3


def _gather_body(tsrc_hbm, tcol_hbm, rowg_hbm, colg_hbm,
                 g1_hbm, g2_hbm, idx_r, idx_c,
                 b1_0, b1_1, b1_2, b2_0, b2_1, b2_2,
                 gs_0, gs_1, gs_2, ws_0, ws_1, ws_2):
    nt = rowg_hbm.shape[1]
    wid = lax.axis_index("c") * NS + lax.axis_index("s")
    base = wid * nt * CHUNK
    bufs1 = (b1_0, b1_1, b1_2)
    bufs2 = (b2_0, b2_1, b2_2)
    gs = (gs_0, gs_1, gs_2)
    ws = (ws_0, ws_1, ws_2)
    pltpu.sync_copy(rowg_hbm.at[wid], idx_r)
    pltpu.sync_copy(colg_hbm.at[wid], idx_c)

    def fire(j, b):
        pltpu.async_copy(tsrc_hbm.at[idx_r.at[j]], bufs1[b], gs[b])
        pltpu.async_copy(tcol_hbm.at[idx_c.at[j]], bufs2[b], gs[b])

    for b in range(_GS):
        fire(b, b)

    def body(g, carry):
        for b in range(_GS):
            j = g * _GS + b

            @pl.when(j < nt)
            def _():
                dst1 = g1_hbm.at[pl.ds(base + j * CHUNK, CHUNK)]
                dst2 = g2_hbm.at[pl.ds(base + j * CHUNK, CHUNK)]
                pltpu.make_async_copy(tsrc_hbm.at[idx_r.at[j]], bufs1[b], gs[b]).wait()
                pltpu.make_async_copy(tcol_hbm.at[idx_c.at[j]], bufs2[b], gs[b]).wait()
                pltpu.async_copy(bufs1[b], dst1, ws[b])
                pltpu.async_copy(bufs2[b], dst2, ws[b])
                pltpu.make_async_copy(bufs1[b], dst1, ws[b]).wait()
                pltpu.make_async_copy(bufs2[b], dst2, ws[b]).wait()

                @pl.when(j + _GS < nt)
                def _():
                    fire(j + _GS, b)

        return carry

    lax.fori_loop(0, (nt + _GS - 1) // _GS, body, 0)


def _sc_gather(tsrc, tcol, rowg, colg):
    nt = rowg.shape[1]
    ne = NW * nt * CHUNK
    f = pl.kernel(
        _gather_body,
        mesh=_mesh(),
        compiler_params=pltpu.CompilerParams(use_tc_tiling_on_sc=True),
        out_type=[
            jax.ShapeDtypeStruct((ne, 2 * F), jnp.float32),
            jax.ShapeDtypeStruct((ne, 2 * F), jnp.float32),
        ],
        scratch_types=(
            [pltpu.VMEM((nt, CHUNK), jnp.int32)] * 2
            + [pltpu.VMEM((CHUNK, 2 * F), jnp.float32)] * (2 * _GS)
            + [pltpu.SemaphoreType.DMA] * (2 * _GS)
        ),
    )
    return f(tsrc, tcol, rowg, colg)


def _edge_body(g1_ref, g2_ref, ea_ref, we1_ref, be1_ref,
               wa_ref, wn1b_ref, bn1b_ref, wea_ref, m_ref):
    f32 = jnp.float32
    g2 = g2_ref[...]
    h0 = jnp.maximum(g1_ref[:, 0:F] + g2[:, 0:F] + ea_ref[...] * wea_ref[...], 0.0)
    h1 = jnp.maximum(jnp.dot(h0, we1_ref[...], preferred_element_type=f32) + be1_ref[...], 0.0)
    m1 = jnp.maximum(jnp.dot(h1, wa_ref[...], preferred_element_type=f32) + g2[:, F:2 * F], 0.0)
    m = jnp.maximum(
        jnp.dot(m1, wn1b_ref[...], preferred_element_type=f32) + bn1b_ref[...], 0.0)
    cols = lax.broadcasted_iota(jnp.int32, (EBLK, F), 1)
    cnt1 = jnp.where(cols == 0, 1.0, 0.0).astype(f32)
    m_ref[...] = jnp.concatenate([m, cnt1], axis=1)


def _edge(g1, g2, ea_pad, We1, be12, W_a, Wn1b, bn1b2, wea2):
    grid = g1.shape[0] // EBLK
    return pl.pallas_call(
        _edge_body,
        grid=(grid,),
        in_specs=[
            pl.BlockSpec((EBLK, 2 * F), lambda i: (i, 0)),
            pl.BlockSpec((EBLK, 2 * F), lambda i: (i, 0)),
            pl.BlockSpec((EBLK, 1), lambda i: (i, 0)),
            pl.BlockSpec((F, F), lambda i: (0, 0)),
            pl.BlockSpec((1, F), lambda i: (0, 0)),
            pl.BlockSpec((F, F), lambda i: (0, 0)),
            pl.BlockSpec((F, F), lambda i: (0, 0)),
            pl.BlockSpec((1, F), lambda i: (0, 0)),
            pl.BlockSpec((1, F), lambda i: (0, 0)),
        ],
        out_specs=pl.BlockSpec((EBLK, 2 * F), lambda i: (i, 0)),
        out_shape=jax.ShapeDtypeStruct((g1.shape[0], 2 * F), jnp.float32),
    )(g1, g2, ea_pad, We1, be12, W_a, Wn1b, bn1b2, wea2)


def _scatter_body(m_hbm, rown_hbm, acc_hbm, idx, mb_0, mb_1,
                  ls_0, ls_1, zs, acc_sh):
    nt = rown_hbm.shape[1]
    c = lax.axis_index("c")
    s = lax.axis_index("s")
    wid = c * NS + s
    mbufs = (mb_0, mb_1)
    ls = (ls_0, ls_1)
    vzero = jnp.zeros((16,), jnp.float32)

    for b in range(2):
        def zrow(i, carry, _b=b):
            def zcol(k, c2):
                mbufs[_b][i, pl.ds(k * 16, 16)] = vzero
                return c2

            lax.fori_loop(0, 2 * F // 16, zcol, 0)
            return carry

        lax.fori_loop(0, CHUNK, zrow, 0)

    zbase = s * ROWS_PER_TILE
    zdsts = [acc_sh.at[pl.ds(zbase + k * CHUNK, CHUNK)] for k in range(4)]
    ztail = acc_sh.at[pl.ds(zbase + 4 * CHUNK, ROWS_PER_TILE - 4 * CHUNK)]
    for k in range(4):
        pltpu.async_copy(mbufs[k % 2], zdsts[k], zs)
    pltpu.async_copy(mbufs[0].at[pl.ds(0, ROWS_PER_TILE - 4 * CHUNK)], ztail, zs)
    pltpu.sync_copy(rown_hbm.at[wid], idx)
    for k in range(4):
        pltpu.make_async_copy(mbufs[k % 2], zdsts[k], zs).wait()
    pltpu.make_async_copy(mbufs[0].at[pl.ds(0, ROWS_PER_TILE - 4 * CHUNK)], ztail, zs).wait()
    plsc.subcore_barrier()

    def load(j, b):
        pltpu.async_copy(m_hbm.at[pl.ds(wid * nt * CHUNK + j * CHUNK, CHUNK)],
                         mbufs[b], ls[b])

    load(0, 0)
    load(1, 1)

    def body(g, carry):
        for b in range(2):
            j = g * 2 + b
            pltpu.make_async_copy(
                m_hbm.at[pl.ds(wid * nt * CHUNK + j * CHUNK, CHUNK)],
                mbufs[b], ls[b]).wait()
            pltpu.sync_copy(mbufs[b], acc_sh.at[idx.at[j]], add=True)

            @pl.when(j + 2 < nt)
            def _():
                load(j + 2, b)

        return carry

    lax.fori_loop(0, nt // 2, body, 0)
    plsc.subcore_barrier()
    pltpu.sync_copy(acc_sh.at[pl.ds(s * ROWS_PER_TILE, ROWS_PER_TILE)],
                    acc_hbm.at[c, pl.ds(s * ROWS_PER_TILE, ROWS_PER_TILE)])


def _sc_scatter(m, rown):
    nt = rown.shape[1]
    f = pl.kernel(
        _scatter_body,
        mesh=_mesh(),
        compiler_params=pltpu.CompilerParams(use_tc_tiling_on_sc=True),
        out_type=[
            jax.ShapeDtypeStruct((NC, N_PAD, 2 * F), jnp.float32),
        ],
        scratch_types=[
            pltpu.VMEM((nt, CHUNK), jnp.int32),
            pltpu.VMEM((CHUNK, 2 * F), jnp.float32),
            pltpu.VMEM((CHUNK, 2 * F), jnp.float32),
            pltpu.SemaphoreType.DMA,
            pltpu.SemaphoreType.DMA,
            pltpu.SemaphoreType.DMA,
            pltpu.VMEM_SHARED((N_PAD, 2 * F), jnp.float32),
        ],
    )
    return f(m, rown)[0]


def _final_body(x_ref, batch_ref, acca_ref, accb_ref, u_ref, wx_ref, wagg_ref,
                wu2_ref, bn2a_ref, wn2b_ref, bn2b_ref, out_ref):
    f32 = jnp.float32
    accw = acca_ref[0] + acca_ref[1] + accb_ref[0] + accb_ref[1]
    denom = jnp.maximum(accw[:, F:F + 1], 1.0)
    agg = accw[:, 0:F] / denom
    gids = lax.broadcasted_iota(jnp.int32, (NBLK, 8), 1)
    onehot = (batch_ref[...] == gids).astype(f32)
    uproj = jnp.dot(u_ref[...], wu2_ref[...], preferred_element_type=f32)
    h2 = jnp.maximum(
        jnp.dot(x_ref[...], wx_ref[...], preferred_element_type=f32)
        + jnp.dot(agg, wagg_ref[...], preferred_element_type=f32)
        + jnp.dot(onehot, uproj, preferred_element_type=f32)
        + bn2a_ref[...], 0.0)
    out_ref[...] = jnp.dot(h2, wn2b_ref[...], preferred_element_type=f32) + bn2b_ref[...]


def _final(x, batch2, acca, accb, u, Wx, Wagg, Wu2, bn2a2, Wn2b, bn2b2):
    grid = N // NBLK
    return pl.pallas_call(
        _final_body,
        grid=(grid,),
        in_specs=[
            pl.BlockSpec((NBLK, 9), lambda i: (i, 0)),
            pl.BlockSpec((NBLK, 1), lambda i: (i, 0)),
            pl.BlockSpec((NC, NBLK, 2 * F), lambda i: (0, i, 0)),
            pl.BlockSpec((NC, NBLK, 2 * F), lambda i: (0, i, 0)),
            pl.BlockSpec((8, F), lambda i: (0, 0)),
            pl.BlockSpec((9, F), lambda i: (0, 0)),
            pl.BlockSpec((F, F), lambda i: (0, 0)),
            pl.BlockSpec((F, F), lambda i: (0, 0)),
            pl.BlockSpec((1, F), lambda i: (0, 0)),
            pl.BlockSpec((F, 1), lambda i: (0, 0)),
            pl.BlockSpec((1, 1), lambda i: (0, 0)),
        ],
        out_specs=pl.BlockSpec((NBLK, 1), lambda i: (i, 0)),
        out_shape=jax.ShapeDtypeStruct((N, 1), jnp.float32),
    )(x, batch2, acca, accb, u, Wx, Wagg, Wu2, bn2a2, Wn2b, bn2b2)


def kernel(x, edge_index, edge_attr, selection, batch, W_sel, b_sel,
           We0, be0, We1, be1, Wef, bef, Wn1a, bn1a, Wn1b, bn1b,
           Wn2a, bn2a, Wn2b, bn2b):
    f32 = jnp.float32
    i32 = jnp.int32

    W_src = We0[0:9]
    W_dst = We0[9:18]
    w_ea = We0[18:19]
    W_u = We0[19:83]
    Wn1a_x = Wn1a[0:9]
    Wn1a_e = Wn1a[9:73]
    W_a = Wef @ Wn1a_e
    b_a = bef @ Wn1a_e + bn1a
    W_cat = jnp.concatenate([W_dst, Wn1a_x], axis=1)
    bcat = jnp.concatenate([jnp.zeros((F,), f32), b_a])[None]
    Wx = Wn2a[0:9]
    Wagg = Wn2a[9:73]
    Wu2 = Wn2a[73:137]

    batch2 = batch[:, None].astype(i32)
    row = edge_index[0]
    col = edge_index[1]
    pad = E_PAD - E
    pad_g = (jnp.arange(pad, dtype=i32) % jnp.int32(N))
    pad_s = jnp.int32(N) + (jnp.arange(pad, dtype=i32) % jnp.int32(TRASH))
    rowg = jnp.concatenate([row, pad_g])
    colg = jnp.concatenate([col, pad_g])
    rown = jnp.concatenate([row, pad_s])
    ea_pad = jnp.concatenate([edge_attr, jnp.zeros((pad, 1), f32)])

    EH = E_PAD // 2
    TH = T // 2

    def half(a, lo, hi):
        return a[lo:hi].reshape(NW, TH, CHUNK)

    tsrc, tcol, u = _prep(x, batch2, selection, W_sel, b_sel[None], W_u,
                          W_src, be0[None], W_cat, bcat)
    g1a, g2a = _sc_gather(tsrc, tcol, half(rowg, 0, EH), half(colg, 0, EH))
    g1b, g2b = _sc_gather(tsrc, tcol, half(rowg, EH, E_PAD), half(colg, EH, E_PAD))
    ma = _edge(g1a, g2a, ea_pad[:EH], We1, be1[None], W_a, Wn1b, bn1b[None], w_ea)
    mb = _edge(g1b, g2b, ea_pad[EH:], We1, be1[None], W_a, Wn1b, bn1b[None], w_ea)
    acca = _sc_scatter(ma, half(rown, 0, EH))
    accb = _sc_scatter(mb, half(rown, EH, E_PAD))
    out = _final(x, batch2, acca, accb, u, Wx, Wagg, Wu2, bn2a[None],
                 Wn2b, bn2b[None])
    return out.reshape(N)


# speedup vs baseline: 1.0185x; 1.0185x over previous
import functools

import jax
import jax.numpy as jnp
from jax import lax
from jax.experimental import pallas as pl
from jax.experimental.pallas import tpu as pltpu
from jax.experimental.pallas import tpu_sc as plsc

N = 10000
E = 320000
F = 64
NC = 2
NS = 16
NW = NC * NS
CHUNK = 128
T = 80
E_PAD = NW * CHUNK * T
PER_W = T * CHUNK
N_PAD = ((N + NS - 1) // NS + 7) // 8 * 8 * NS
ROWS_PER_TILE = N_PAD // NS
TRASH = N_PAD - N

NBLK = 1000
EBLK = 4096

@functools.cache
def _mesh():
    return plsc.VectorSubcoreMesh(core_axis_name="c", subcore_axis_name="s")


def _prep_body(x_ref, batch_ref, sel_ref, wsel_ref, bsel_ref, wu_ref,
               wsrc_ref, be0_ref, wcat_ref, bcat_ref,
               tsrc_ref, tcol_ref, u_ref):
    f32 = jnp.float32
    u_val = jnp.dot(sel_ref[...], wsel_ref[...], preferred_element_type=f32) + bsel_ref[...]
    ug = jnp.dot(u_val, wu_ref[...], preferred_element_type=f32)
    xb = x_ref[...]
    gids = lax.broadcasted_iota(jnp.int32, (NBLK, 8), 1)
    onehot = (batch_ref[...] == gids).astype(f32)
    ts = (jnp.dot(xb, wsrc_ref[...], preferred_element_type=f32)
          + jnp.dot(onehot, ug, preferred_element_type=f32)
          + be0_ref[...])
    tsrc_ref[...] = jnp.concatenate([ts, jnp.zeros((NBLK, F), f32)], axis=1)
    tcol_ref[...] = jnp.dot(xb, wcat_ref[...], preferred_element_type=f32) + bcat_ref[...]
    u_ref[...] = u_val


def _prep(x, batch2, selection, W_sel, b_sel2, W_u, W_src, be02, W_cat, bcat2):
    f32 = jnp.float32
    grid = N // NBLK
    return pl.pallas_call(
        _prep_body,
        grid=(grid,),
        in_specs=[
            pl.BlockSpec((NBLK, 9), lambda i: (i, 0)),
            pl.BlockSpec((NBLK, 1), lambda i: (i, 0)),
            pl.BlockSpec((8, 512), lambda i: (0, 0)),
            pl.BlockSpec((512, F), lambda i: (0, 0)),
            pl.BlockSpec((1, F), lambda i: (0, 0)),
            pl.BlockSpec((F, F), lambda i: (0, 0)),
            pl.BlockSpec((9, F), lambda i: (0, 0)),
            pl.BlockSpec((1, F), lambda i: (0, 0)),
            pl.BlockSpec((9, 2 * F), lambda i: (0, 0)),
            pl.BlockSpec((1, 2 * F), lambda i: (0, 0)),
        ],
        out_specs=[
            pl.BlockSpec((NBLK, 2 * F), lambda i: (i, 0)),
            pl.BlockSpec((NBLK, 2 * F), lambda i: (i, 0)),
            pl.BlockSpec((8, F), lambda i: (0, 0)),
        ],
        out_shape=[
            jax.ShapeDtypeStruct((N, 2 * F), f32),
            jax.ShapeDtypeStruct((N, 2 * F), f32),
            jax.ShapeDtypeStruct((8, F), f32),
        ],
    )(x, batch2, selection, W_sel, b_sel2, W_u, W_src, be02, W_cat, bcat2)


_GS = 3


def _gather_body(tsrc_hbm, tcol_hbm, rowg_hbm, colg_hbm,
                 g1_hbm, g2_hbm, idx_r, idx_c,
                 b1_0, b1_1, b1_2, b2_0, b2_1, b2_2,
                 gs_0, gs_1, gs_2, ws_0, ws_1, ws_2):
    nt = rowg_hbm.shape[1]
    wid = lax.axis_index("c") * NS + lax.axis_index("s")
    base = wid * nt * CHUNK
    bufs1 = (b1_0, b1_1, b1_2)
    bufs2 = (b2_0, b2_1, b2_2)
    gs = (gs_0, gs_1, gs_2)
    ws = (ws_0, ws_1, ws_2)
    pltpu.sync_copy(rowg_hbm.at[wid], idx_r)
    pltpu.sync_copy(colg_hbm.at[wid], idx_c)

    def fire(j, b):
        pltpu.async_copy(tsrc_hbm.at[idx_r.at[j]], bufs1[b], gs[b])
        pltpu.async_copy(tcol_hbm.at[idx_c.at[j]], bufs2[b], gs[b])

    for b in range(_GS):
        fire(b, b)

    def body(g, carry):
        for b in range(_GS):
            j = g * _GS + b

            @pl.when(j < nt)
            def _():
                dst1 = g1_hbm.at[pl.ds(base + j * CHUNK, CHUNK)]
                dst2 = g2_hbm.at[pl.ds(base + j * CHUNK, CHUNK)]
                pltpu.make_async_copy(tsrc_hbm.at[idx_r.at[j]], bufs1[b], gs[b]).wait()
                pltpu.make_async_copy(tcol_hbm.at[idx_c.at[j]], bufs2[b], gs[b]).wait()
                pltpu.async_copy(bufs1[b], dst1, ws[b])
                pltpu.async_copy(bufs2[b], dst2, ws[b])
                pltpu.make_async_copy(bufs1[b], dst1, ws[b]).wait()
                pltpu.make_async_copy(bufs2[b], dst2, ws[b]).wait()

                @pl.when(j + _GS < nt)
                def _():
                    fire(j + _GS, b)

        return carry

    lax.fori_loop(0, (nt + _GS - 1) // _GS, body, 0)


def _sc_gather(tsrc, tcol, rowg, colg):
    nt = rowg.shape[1]
    ne = NW * nt * CHUNK
    f = pl.kernel(
        _gather_body,
        mesh=_mesh(),
        out_type=[
            jax.ShapeDtypeStruct((ne, 2 * F), jnp.float32),
            jax.ShapeDtypeStruct((ne, 2 * F), jnp.float32),
        ],
        scratch_types=(
            [pltpu.VMEM((nt, CHUNK), jnp.int32)] * 2
            + [pltpu.VMEM((CHUNK, 2 * F), jnp.float32)] * (2 * _GS)
            + [pltpu.SemaphoreType.DMA] * (2 * _GS)
        ),
    )
    return f(tsrc, tcol, rowg, colg)


def _edge_body(g1_ref, g2_ref, ea_ref, we1_ref, be1_ref,
               wa_ref, wn1b_ref, bn1b_ref, wea_ref, m_ref):
    f32 = jnp.float32
    g1 = g1_ref[...]
    g2 = g2_ref[...]
    h0 = jnp.maximum(g1[:, 0:F] + g2[:, 0:F] + ea_ref[...] * wea_ref[...], 0.0)
    h1 = jnp.maximum(jnp.dot(h0, we1_ref[...], preferred_element_type=f32) + be1_ref[...], 0.0)
    m1 = jnp.maximum(jnp.dot(h1, wa_ref[...], preferred_element_type=f32) + g2[:, F:2 * F], 0.0)
    m = jnp.maximum(
        jnp.dot(m1, wn1b_ref[...], preferred_element_type=f32) + bn1b_ref[...], 0.0)
    cols = lax.broadcasted_iota(jnp.int32, (EBLK, F), 1)
    cnt1 = jnp.where(cols == 0, 1.0, 0.0).astype(f32)
    m_ref[...] = jnp.concatenate([m, cnt1], axis=1)


def _edge(g1, g2, ea, We1, be12, W_a, Wn1b, bn1b2, wea2):
    grid = g1.shape[0] // EBLK
    return pl.pallas_call(
        _edge_body,
        grid=(grid,),
        in_specs=[
            pl.BlockSpec((EBLK, 2 * F), lambda i: (i, 0)),
            pl.BlockSpec((EBLK, 2 * F), lambda i: (i, 0)),
            pl.BlockSpec((EBLK, 1), lambda i: (i, 0)),
            pl.BlockSpec((F, F), lambda i: (0, 0)),
            pl.BlockSpec((1, F), lambda i: (0, 0)),
            pl.BlockSpec((F, F), lambda i: (0, 0)),
            pl.BlockSpec((F, F), lambda i: (0, 0)),
            pl.BlockSpec((1, F), lambda i: (0, 0)),
            pl.BlockSpec((1, F), lambda i: (0, 0)),
        ],
        out_specs=pl.BlockSpec((EBLK, 2 * F), lambda i: (i, 0)),
        out_shape=jax.ShapeDtypeStruct((g1.shape[0], 2 * F), jnp.float32),
    )(g1, g2, ea, We1, be12, W_a, Wn1b, bn1b2, wea2)


def _scatter_body(m_hbm, rown_hbm, acc_hbm, idx, mb_0, mb_1,
                  ls_0, ls_1, zs, acc_sh):
    nt = rown_hbm.shape[1]
    c = lax.axis_index("c")
    s = lax.axis_index("s")
    wid = c * NS + s
    mbufs = (mb_0, mb_1)
    ls = (ls_0, ls_1)
    vzero = jnp.zeros((16,), jnp.float32)

    for b in range(2):
        def zrow(i, carry, _b=b):
            def zcol(k, c2):
                mbufs[_b][i, pl.ds(k * 16, 16)] = vzero
                return c2

            lax.fori_loop(0, 2 * F // 16, zcol, 0)
            return carry

        lax.fori_loop(0, CHUNK, zrow, 0)

    zbase = s * ROWS_PER_TILE
    zdsts = [acc_sh.at[pl.ds(zbase + k * CHUNK, CHUNK)] for k in range(4)]
    ztail = acc_sh.at[pl.ds(zbase + 4 * CHUNK, ROWS_PER_TILE - 4 * CHUNK)]
    for k in range(4):
        pltpu.async_copy(mbufs[k % 2], zdsts[k], zs)
    pltpu.async_copy(mbufs[0].at[pl.ds(0, ROWS_PER_TILE - 4 * CHUNK)], ztail, zs)
    pltpu.sync_copy(rown_hbm.at[wid], idx)
    for k in range(4):
        pltpu.make_async_copy(mbufs[k % 2], zdsts[k], zs).wait()
    pltpu.make_async_copy(mbufs[0].at[pl.ds(0, ROWS_PER_TILE - 4 * CHUNK)], ztail, zs).wait()
    plsc.subcore_barrier()

    def load(j, b):
        pltpu.async_copy(m_hbm.at[pl.ds(wid * nt * CHUNK + j * CHUNK, CHUNK)],
                         mbufs[b], ls[b])

    load(0, 0)
    load(1, 1)

    def body(g, carry):
        for b in range(2):
            j = g * 2 + b
            pltpu.make_async_copy(
                m_hbm.at[pl.ds(wid * nt * CHUNK + j * CHUNK, CHUNK)],
                mbufs[b], ls[b]).wait()
            pltpu.sync_copy(mbufs[b], acc_sh.at[idx.at[j]], add=True)

            @pl.when(j + 2 < nt)
            def _():
                load(j + 2, b)

        return carry

    lax.fori_loop(0, nt // 2, body, 0)
    plsc.subcore_barrier()
    pltpu.sync_copy(acc_sh.at[pl.ds(s * ROWS_PER_TILE, ROWS_PER_TILE)],
                    acc_hbm.at[c, pl.ds(s * ROWS_PER_TILE, ROWS_PER_TILE)])


def _sc_scatter(m, rown):
    nt = rown.shape[1]
    f = pl.kernel(
        _scatter_body,
        mesh=_mesh(),
        compiler_params=pltpu.CompilerParams(use_tc_tiling_on_sc=True),
        out_type=[
            jax.ShapeDtypeStruct((NC, N_PAD, 2 * F), jnp.float32),
        ],
        scratch_types=[
            pltpu.VMEM((nt, CHUNK), jnp.int32),
            pltpu.VMEM((CHUNK, 2 * F), jnp.float32),
            pltpu.VMEM((CHUNK, 2 * F), jnp.float32),
            pltpu.SemaphoreType.DMA,
            pltpu.SemaphoreType.DMA,
            pltpu.SemaphoreType.DMA,
            pltpu.VMEM_SHARED((N_PAD, 2 * F), jnp.float32),
        ],
    )
    return f(m, rown)[0]


def _final_body(x_ref, batch_ref, acca_ref, accb_ref, u_ref, wx_ref, wagg_ref,
                wu2_ref, bn2a_ref, wn2b_ref, bn2b_ref, out_ref):
    f32 = jnp.float32
    accw = acca_ref[0] + acca_ref[1] + accb_ref[0] + accb_ref[1]
    denom = jnp.maximum(accw[:, F:F + 1], 1.0)
    agg = accw[:, 0:F] / denom
    gids = lax.broadcasted_iota(jnp.int32, (NBLK, 8), 1)
    onehot = (batch_ref[...] == gids).astype(f32)
    uproj = jnp.dot(u_ref[...], wu2_ref[...], preferred_element_type=f32)
    h2 = jnp.maximum(
        jnp.dot(x_ref[...], wx_ref[...], preferred_element_type=f32)
        + jnp.dot(agg, wagg_ref[...], preferred_element_type=f32)
        + jnp.dot(onehot, uproj, preferred_element_type=f32)
        + bn2a_ref[...], 0.0)
    out_ref[...] = jnp.dot(h2, wn2b_ref[...], preferred_element_type=f32) + bn2b_ref[...]


def _final(x, batch2, acca, accb, u, Wx, Wagg, Wu2, bn2a2, Wn2b, bn2b2):
    grid = N // NBLK
    return pl.pallas_call(
        _final_body,
        grid=(grid,),
        in_specs=[
            pl.BlockSpec((NBLK, 9), lambda i: (i, 0)),
            pl.BlockSpec((NBLK, 1), lambda i: (i, 0)),
            pl.BlockSpec((NC, NBLK, 2 * F), lambda i: (0, i, 0)),
            pl.BlockSpec((NC, NBLK, 2 * F), lambda i: (0, i, 0)),
            pl.BlockSpec((8, F), lambda i: (0, 0)),
            pl.BlockSpec((9, F), lambda i: (0, 0)),
            pl.BlockSpec((F, F), lambda i: (0, 0)),
            pl.BlockSpec((F, F), lambda i: (0, 0)),
            pl.BlockSpec((1, F), lambda i: (0, 0)),
            pl.BlockSpec((F, 1), lambda i: (0, 0)),
            pl.BlockSpec((1, 1), lambda i: (0, 0)),
        ],
        out_specs=pl.BlockSpec((NBLK, 1), lambda i: (i, 0)),
        out_shape=jax.ShapeDtypeStruct((N, 1), jnp.float32),
    )(x, batch2, acca, accb, u, Wx, Wagg, Wu2, bn2a2, Wn2b, bn2b2)


def kernel(x, edge_index, edge_attr, selection, batch, W_sel, b_sel,
           We0, be0, We1, be1, Wef, bef, Wn1a, bn1a, Wn1b, bn1b,
           Wn2a, bn2a, Wn2b, bn2b):
    f32 = jnp.float32
    i32 = jnp.int32

    W_src = We0[0:9]
    W_dst = We0[9:18]
    w_ea = We0[18:19]
    W_u = We0[19:83]
    Wn1a_x = Wn1a[0:9]
    Wn1a_e = Wn1a[9:73]
    W_a = Wef @ Wn1a_e
    b_a = bef @ Wn1a_e + bn1a
    W_cat = jnp.concatenate([W_dst, Wn1a_x], axis=1)
    bcat = jnp.concatenate([jnp.zeros((F,), f32), b_a])[None]
    Wx = Wn2a[0:9]
    Wagg = Wn2a[9:73]
    Wu2 = Wn2a[73:137]

    batch2 = batch[:, None].astype(i32)
    row = edge_index[0]
    col = edge_index[1]
    pad = E_PAD - E
    pad_g = (jnp.arange(pad, dtype=i32) % jnp.int32(N))
    pad_s = jnp.int32(N) + (jnp.arange(pad, dtype=i32) % jnp.int32(TRASH))
    rowg = jnp.concatenate([row, pad_g])
    colg = jnp.concatenate([col, pad_g])
    rown = jnp.concatenate([row, pad_s])

    EH = E_PAD // 2
    TH = T // 2

    def half(a, lo, hi):
        return a[lo:hi].reshape(NW, TH, CHUNK)

    tsrc, tcol, u = _prep(x, batch2, selection, W_sel, b_sel[None], W_u,
                          W_src, be0[None], W_cat, bcat)
    g1a, g2a = _sc_gather(tsrc, tcol, half(rowg, 0, EH), half(colg, 0, EH))
    g1b, g2b = _sc_gather(tsrc, tcol, half(rowg, EH, E_PAD), half(colg, EH, E_PAD))
    ea_b = jnp.concatenate([edge_attr[EH:], jnp.zeros((pad, 1), f32)])
    ma = _edge(g1a, g2a, edge_attr, We1, be1[None], W_a, Wn1b, bn1b[None], w_ea)
    mb = _edge(g1b, g2b, ea_b, We1, be1[None], W_a, Wn1b, bn1b[None], w_ea)
    acca = _sc_scatter(ma, half(rown, 0, EH))
    accb = _sc_scatter(mb, half(rown, EH, E_PAD))
    out = _final(x, batch2, acca, accb, u, Wx, Wagg, Wu2, bn2a[None],
                 Wn2b, bn2b[None])
    return out.reshape(N)


# speedup vs baseline: 1.1505x; 1.1297x over previous
import functools

import jax
import jax.numpy as jnp
from jax import lax
from jax.experimental import pallas as pl
from jax.experimental.pallas import tpu as pltpu
from jax.experimental.pallas import tpu_sc as plsc

N = 10000
E = 320000
F = 64
NC = 2
NS = 16
NW = NC * NS
CHUNK = 128
T = 80
E_PAD = NW * CHUNK * T
PER_W = T * CHUNK
N_PAD = ((N + NS - 1) // NS + 7) // 8 * 8 * NS
ROWS_PER_TILE = N_PAD // NS
TRASH = N_PAD - N

NBLK = 1000
EBLK = 4096

@functools.cache
def _mesh():
    return plsc.VectorSubcoreMesh(core_axis_name="c", subcore_axis_name="s")


def _prep_body(x_ref, batch_ref, sel_ref, wsel_ref, bsel_ref, wu_ref,
               wsrc_ref, be0_ref, wcat_ref, bcat_ref,
               tsrc_ref, tcol_ref, u_ref):
    f32 = jnp.float32
    u_val = jnp.dot(sel_ref[...], wsel_ref[...], preferred_element_type=f32) + bsel_ref[...]
    ug = jnp.dot(u_val, wu_ref[...], preferred_element_type=f32)
    xb = x_ref[...]
    gids = lax.broadcasted_iota(jnp.int32, (NBLK, 8), 1)
    onehot = (batch_ref[...] == gids).astype(f32)
    ts = (jnp.dot(xb, wsrc_ref[...], preferred_element_type=f32)
          + jnp.dot(onehot, ug, preferred_element_type=f32)
          + be0_ref[...])
    tsrc_ref[...] = jnp.concatenate([ts, jnp.zeros((NBLK, F), f32)], axis=1)
    tcol_ref[...] = jnp.dot(xb, wcat_ref[...], preferred_element_type=f32) + bcat_ref[...]
    u_ref[...] = u_val


def _prep(x, batch2, selection, W_sel, b_sel2, W_u, W_src, be02, W_cat, bcat2):
    f32 = jnp.float32
    grid = N // NBLK
    return pl.pallas_call(
        _prep_body,
        grid=(grid,),
        in_specs=[
            pl.BlockSpec((NBLK, 9), lambda i: (i, 0)),
            pl.BlockSpec((NBLK, 1), lambda i: (i, 0)),
            pl.BlockSpec((8, 512), lambda i: (0, 0)),
            pl.BlockSpec((512, F), lambda i: (0, 0)),
            pl.BlockSpec((1, F), lambda i: (0, 0)),
            pl.BlockSpec((F, F), lambda i: (0, 0)),
            pl.BlockSpec((9, F), lambda i: (0, 0)),
            pl.BlockSpec((1, F), lambda i: (0, 0)),
            pl.BlockSpec((9, 2 * F), lambda i: (0, 0)),
            pl.BlockSpec((1, 2 * F), lambda i: (0, 0)),
        ],
        out_specs=[
            pl.BlockSpec((NBLK, 2 * F), lambda i: (i, 0)),
            pl.BlockSpec((NBLK, 2 * F), lambda i: (i, 0)),
            pl.BlockSpec((8, F), lambda i: (0, 0)),
        ],
        out_shape=[
            jax.ShapeDtypeStruct((N, 2 * F), f32),
            jax.ShapeDtypeStruct((N, 2 * F), f32),
            jax.ShapeDtypeStruct((8, F), f32),
        ],
    )(x, batch2, selection, W_sel, b_sel2, W_u, W_src, be02, W_cat, bcat2)


_GS = 3


def _gather_body(tsrc_hbm, tcol_hbm, rowg_hbm, colg_hbm,
                 g1_hbm, g2_hbm, idx_r, idx_c,
                 b1_0, b1_1, b1_2, b2_0, b2_1, b2_2,
                 gs_0, gs_1, gs_2, ws_0, ws_1, ws_2):
    nt = rowg_hbm.shape[1]
    wid = lax.axis_index("c") * NS + lax.axis_index("s")
    base = wid * nt * CHUNK
    bufs1 = (b1_0, b1_1, b1_2)
    bufs2 = (b2_0, b2_1, b2_2)
    gs = (gs_0, gs_1, gs_2)
    ws = (ws_0, ws_1, ws_2)
    pltpu.sync_copy(rowg_hbm.at[wid], idx_r)
    pltpu.sync_copy(colg_hbm.at[wid], idx_c)

    def fire(j, b):
        pltpu.async_copy(tsrc_hbm.at[idx_r.at[j]], bufs1[b], gs[b])
        pltpu.async_copy(tcol_hbm.at[idx_c.at[j]], bufs2[b], gs[b])

    for b in range(_GS):
        fire(b, b)

    def body(g, carry):
        for b in range(_GS):
            j = g * _GS + b

            @pl.when(j < nt)
            def _():
                dst1 = g1_hbm.at[pl.ds(base + j * CHUNK, CHUNK)]
                dst2 = g2_hbm.at[pl.ds(base + j * CHUNK, CHUNK)]
                pltpu.make_async_copy(tsrc_hbm.at[idx_r.at[j]], bufs1[b], gs[b]).wait()
                pltpu.make_async_copy(tcol_hbm.at[idx_c.at[j]], bufs2[b], gs[b]).wait()
                pltpu.async_copy(bufs1[b], dst1, ws[b])
                pltpu.async_copy(bufs2[b], dst2, ws[b])
                pltpu.make_async_copy(bufs1[b], dst1, ws[b]).wait()
                pltpu.make_async_copy(bufs2[b], dst2, ws[b]).wait()

                @pl.when(j + _GS < nt)
                def _():
                    fire(j + _GS, b)

        return carry

    lax.fori_loop(0, (nt + _GS - 1) // _GS, body, 0)


def _sc_gather(tsrc, tcol, rowg, colg):
    nt = rowg.shape[1]
    ne = NW * nt * CHUNK
    f = pl.kernel(
        _gather_body,
        mesh=_mesh(),
        out_type=[
            jax.ShapeDtypeStruct((ne, 2 * F), jnp.float32),
            jax.ShapeDtypeStruct((ne, 2 * F), jnp.float32),
        ],
        scratch_types=(
            [pltpu.VMEM((nt, CHUNK), jnp.int32)] * 2
            + [pltpu.VMEM((CHUNK, 2 * F), jnp.float32)] * (2 * _GS)
            + [pltpu.SemaphoreType.DMA] * (2 * _GS)
        ),
    )
    return f(tsrc, tcol, rowg, colg)


def _edge_body(g1_ref, g2_ref, ea_ref, we1_ref, be1_ref,
               wa_ref, wn1b_ref, bn1b_ref, wea_ref, m_ref):
    f32 = jnp.float32
    g1 = g1_ref[...]
    g2 = g2_ref[...]
    h0 = jnp.maximum(g1[:, 0:F] + g2[:, 0:F] + ea_ref[...] * wea_ref[...], 0.0)
    h1 = jnp.maximum(jnp.dot(h0, we1_ref[...], preferred_element_type=f32) + be1_ref[...], 0.0)
    m1 = jnp.maximum(jnp.dot(h1, wa_ref[...], preferred_element_type=f32) + g2[:, F:2 * F], 0.0)
    m = jnp.maximum(
        jnp.dot(m1, wn1b_ref[...], preferred_element_type=f32) + bn1b_ref[...], 0.0)
    cols = lax.broadcasted_iota(jnp.int32, (EBLK, F), 1)
    cnt1 = jnp.where(cols == 0, 1.0, 0.0).astype(f32)
    m_ref[...] = jnp.concatenate([m, cnt1], axis=1)


def _edge(g1, g2, ea_col, off, We1, be12, W_a, Wn1b, bn1b2, wea2):
    grid = g1.shape[0] // EBLK
    return pl.pallas_call(
        _edge_body,
        grid=(grid,),
        in_specs=[
            pl.BlockSpec((EBLK, 2 * F), lambda i: (i, 0)),
            pl.BlockSpec((EBLK, 2 * F), lambda i: (i, 0)),
            pl.BlockSpec((EBLK, 1), lambda i: (i + off, 0)),
            pl.BlockSpec((F, F), lambda i: (0, 0)),
            pl.BlockSpec((1, F), lambda i: (0, 0)),
            pl.BlockSpec((F, F), lambda i: (0, 0)),
            pl.BlockSpec((F, F), lambda i: (0, 0)),
            pl.BlockSpec((1, F), lambda i: (0, 0)),
            pl.BlockSpec((1, F), lambda i: (0, 0)),
        ],
        out_specs=pl.BlockSpec((EBLK, 2 * F), lambda i: (i, 0)),
        out_shape=jax.ShapeDtypeStruct((g1.shape[0], 2 * F), jnp.float32),
    )(g1, g2, ea_col, We1, be12, W_a, Wn1b, bn1b2, wea2)


def _scatter_body(m_hbm, rown_hbm, acc_hbm, idx, mb_0, mb_1,
                  ls_0, ls_1, zs, acc_sh):
    nt = rown_hbm.shape[1]
    c = lax.axis_index("c")
    s = lax.axis_index("s")
    wid = c * NS + s
    mbufs = (mb_0, mb_1)
    ls = (ls_0, ls_1)
    vzero = jnp.zeros((16,), jnp.float32)

    for b in range(2):
        def zrow(i, carry, _b=b):
            def zcol(k, c2):
                mbufs[_b][i, pl.ds(k * 16, 16)] = vzero
                return c2

            lax.fori_loop(0, 2 * F // 16, zcol, 0)
            return carry

        lax.fori_loop(0, CHUNK, zrow, 0)

    zbase = s * ROWS_PER_TILE
    zdsts = [acc_sh.at[pl.ds(zbase + k * CHUNK, CHUNK)] for k in range(4)]
    ztail = acc_sh.at[pl.ds(zbase + 4 * CHUNK, ROWS_PER_TILE - 4 * CHUNK)]
    for k in range(4):
        pltpu.async_copy(mbufs[k % 2], zdsts[k], zs)
    pltpu.async_copy(mbufs[0].at[pl.ds(0, ROWS_PER_TILE - 4 * CHUNK)], ztail, zs)
    pltpu.sync_copy(rown_hbm.at[wid], idx)
    for k in range(4):
        pltpu.make_async_copy(mbufs[k % 2], zdsts[k], zs).wait()
    pltpu.make_async_copy(mbufs[0].at[pl.ds(0, ROWS_PER_TILE - 4 * CHUNK)], ztail, zs).wait()
    plsc.subcore_barrier()

    def load(j, b):
        pltpu.async_copy(m_hbm.at[pl.ds(wid * nt * CHUNK + j * CHUNK, CHUNK)],
                         mbufs[b], ls[b])

    load(0, 0)
    load(1, 1)

    def body(g, carry):
        for b in range(2):
            j = g * 2 + b
            pltpu.make_async_copy(
                m_hbm.at[pl.ds(wid * nt * CHUNK + j * CHUNK, CHUNK)],
                mbufs[b], ls[b]).wait()
            pltpu.sync_copy(mbufs[b], acc_sh.at[idx.at[j]], add=True)

            @pl.when(j + 2 < nt)
            def _():
                load(j + 2, b)

        return carry

    lax.fori_loop(0, nt // 2, body, 0)
    plsc.subcore_barrier()
    pltpu.sync_copy(acc_sh.at[pl.ds(s * ROWS_PER_TILE, ROWS_PER_TILE)],
                    acc_hbm.at[c, pl.ds(s * ROWS_PER_TILE, ROWS_PER_TILE)])


def _sc_scatter(m, rown):
    nt = rown.shape[1]
    f = pl.kernel(
        _scatter_body,
        mesh=_mesh(),
        compiler_params=pltpu.CompilerParams(use_tc_tiling_on_sc=True),
        out_type=[
            jax.ShapeDtypeStruct((NC, N_PAD, 2 * F), jnp.float32),
        ],
        scratch_types=[
            pltpu.VMEM((nt, CHUNK), jnp.int32),
            pltpu.VMEM((CHUNK, 2 * F), jnp.float32),
            pltpu.VMEM((CHUNK, 2 * F), jnp.float32),
            pltpu.SemaphoreType.DMA,
            pltpu.SemaphoreType.DMA,
            pltpu.SemaphoreType.DMA,
            pltpu.VMEM_SHARED((N_PAD, 2 * F), jnp.float32),
        ],
    )
    return f(m, rown)[0]


def _final_body(x_ref, batch_ref, acca_ref, accb_ref, u_ref, wx_ref, wagg_ref,
                wu2_ref, bn2a_ref, wn2b_ref, bn2b_ref, out_ref):
    f32 = jnp.float32
    accw = acca_ref[0] + acca_ref[1] + accb_ref[0] + accb_ref[1]
    denom = jnp.maximum(accw[:, F:F + 1], 1.0)
    agg = accw[:, 0:F] / denom
    gids = lax.broadcasted_iota(jnp.int32, (NBLK, 8), 1)
    onehot = (batch_ref[...] == gids).astype(f32)
    uproj = jnp.dot(u_ref[...], wu2_ref[...], preferred_element_type=f32)
    h2 = jnp.maximum(
        jnp.dot(x_ref[...], wx_ref[...], preferred_element_type=f32)
        + jnp.dot(agg, wagg_ref[...], preferred_element_type=f32)
        + jnp.dot(onehot, uproj, preferred_element_type=f32)
        + bn2a_ref[...], 0.0)
    out_ref[...] = jnp.dot(h2, wn2b_ref[...], preferred_element_type=f32) + bn2b_ref[...]


def _final(x, batch2, acca, accb, u, Wx, Wagg, Wu2, bn2a2, Wn2b, bn2b2):
    grid = N // NBLK
    return pl.pallas_call(
        _final_body,
        grid=(grid,),
        in_specs=[
            pl.BlockSpec((NBLK, 9), lambda i: (i, 0)),
            pl.BlockSpec((NBLK, 1), lambda i: (i, 0)),
            pl.BlockSpec((NC, NBLK, 2 * F), lambda i: (0, i, 0)),
            pl.BlockSpec((NC, NBLK, 2 * F), lambda i: (0, i, 0)),
            pl.BlockSpec((8, F), lambda i: (0, 0)),
            pl.BlockSpec((9, F), lambda i: (0, 0)),
            pl.BlockSpec((F, F), lambda i: (0, 0)),
            pl.BlockSpec((F, F), lambda i: (0, 0)),
            pl.BlockSpec((1, F), lambda i: (0, 0)),
            pl.BlockSpec((F, 1), lambda i: (0, 0)),
            pl.BlockSpec((1, 1), lambda i: (0, 0)),
        ],
        out_specs=pl.BlockSpec((NBLK, 1), lambda i: (i, 0)),
        out_shape=jax.ShapeDtypeStruct((N, 1), jnp.float32),
    )(x, batch2, acca, accb, u, Wx, Wagg, Wu2, bn2a2, Wn2b, bn2b2)


def kernel(x, edge_index, edge_attr, selection, batch, W_sel, b_sel,
           We0, be0, We1, be1, Wef, bef, Wn1a, bn1a, Wn1b, bn1b,
           Wn2a, bn2a, Wn2b, bn2b):
    f32 = jnp.float32
    i32 = jnp.int32

    W_src = We0[0:9]
    W_dst = We0[9:18]
    w_ea = We0[18:19]
    W_u = We0[19:83]
    Wn1a_x = Wn1a[0:9]
    Wn1a_e = Wn1a[9:73]
    W_a = Wef @ Wn1a_e
    b_a = bef @ Wn1a_e + bn1a
    W_cat = jnp.concatenate([W_dst, Wn1a_x], axis=1)
    bcat = jnp.concatenate([jnp.zeros((F,), f32), b_a])[None]
    Wx = Wn2a[0:9]
    Wagg = Wn2a[9:73]
    Wu2 = Wn2a[73:137]

    batch2 = batch[:, None].astype(i32)
    row = edge_index[0]
    col = edge_index[1]
    pad = E_PAD - E
    pad_g = (jnp.arange(pad, dtype=i32) % jnp.int32(N))
    pad_s = jnp.int32(N) + (jnp.arange(pad, dtype=i32) % jnp.int32(TRASH))
    rowg = jnp.concatenate([row, pad_g])
    colg = jnp.concatenate([col, pad_g])
    rown = jnp.concatenate([row, pad_s])
    ea_pad = jnp.concatenate([edge_attr[:, 0], jnp.zeros((pad,), f32)])

    EH = E_PAD // 2
    TH = T // 2

    def half(a, lo, hi):
        return a[lo:hi].reshape(NW, TH, CHUNK)

    tsrc, tcol, u = _prep(x, batch2, selection, W_sel, b_sel[None], W_u,
                          W_src, be0[None], W_cat, bcat)
    ea_col = ea_pad.reshape(E_PAD, 1)
    g1a, g2a = _sc_gather(tsrc, tcol, half(rowg, 0, EH), half(colg, 0, EH))
    g1b, g2b = _sc_gather(tsrc, tcol, half(rowg, EH, E_PAD), half(colg, EH, E_PAD))
    ma = _edge(g1a, g2a, ea_col, 0, We1, be1[None], W_a, Wn1b, bn1b[None], w_ea)
    mb = _edge(g1b, g2b, ea_col, EH // EBLK, We1, be1[None], W_a, Wn1b, bn1b[None], w_ea)
    acca = _sc_scatter(ma, half(rown, 0, EH))
    accb = _sc_scatter(mb, half(rown, EH, E_PAD))
    out = _final(x, batch2, acca, accb, u, Wx, Wagg, Wu2, bn2a[None],
                 Wn2b, bn2b[None])
    return out.reshape(N)
